# baseline (speedup 1.0000x reference)
import functools

import jax
import jax.numpy as jnp
from jax import lax
from jax.experimental import pallas as pl
from jax.experimental.pallas import tpu as pltpu
from jax.experimental.pallas import tpu_sc as plsc

NUM_TAGS = 100000
D = 32
B = 4096
H = 50
L = 16
NC, NS = 2, 16
NW = NC * NS
RPW = B // NW
NBUF = 8
SET_BYTES = 2 * H * D * 4


def _sc_kernel(table_hbm, ct_hbm, ui_hbm, out_hbm,
               cidx, uidx, outv, cbufs, ubufs, sems):
    wid = lax.axis_index("s") * NC + lax.axis_index("c")
    row0 = wid * RPW

    pltpu.sync_copy(ct_hbm.at[pl.ds(row0, RPW)], cidx)
    pltpu.sync_copy(ui_hbm.at[pl.ds(row0, RPW)], uidx)

    lane = lax.broadcasted_iota(jnp.int32, (L,), 0)
    last_lane = lane == (L - 1)

    def start(j, b):
        pltpu.async_copy(table_hbm.at[cidx.at[j]], cbufs[b], sems[b])
        pltpu.async_copy(table_hbm.at[uidx.at[j]], ubufs[b], sems[b])

    def compute(j, b):
        cb, ub = cbufs[b], ubufs[b]
        c0 = jnp.zeros((L,), jnp.float32)
        c1 = jnp.zeros((L,), jnp.float32)
        u0 = jnp.zeros((L,), jnp.float32)
        u1 = jnp.zeros((L,), jnp.float32)
        for r in range(H):
            c0 = c0 + cb[r, pl.ds(0, L)]
            c1 = c1 + cb[r, pl.ds(L, L)]
            u0 = u0 + ub[r, pl.ds(0, L)]
            u1 = u1 + ub[r, pl.ds(L, L)]
        cs = plsc.cumsum(c0 * u0 + c1 * u1) * (1.0 / (H * H))
        row_idx = jnp.full((L,), j, jnp.int32)
        plsc.store_scatter(outv, [row_idx], cs, mask=last_lane)

    def step(j, b, do_start):
        pltpu.make_async_copy(table_hbm.at[cidx.at[0]], cbufs[b], sems[b]).wait()
        pltpu.make_async_copy(table_hbm.at[uidx.at[0]], ubufs[b], sems[b]).wait()
        if do_start:
            start(j + NBUF - 1, (b + NBUF - 1) % NBUF)
        compute(j, b)

    for p in range(NBUF - 1):
        start(p, p)

    @pl.loop(0, RPW - NBUF, step=NBUF)
    def _chunks(j):
        for b in range(NBUF):
            step(j + b, b, True)

    for b in range(NBUF):
        step(RPW - NBUF + b, b, b == 0)

    pltpu.sync_copy(outv, out_hbm.at[pl.ds(row0, RPW)])


@jax.jit
def kernel(course_tags, user_interests, tag_embedding):
    ct = course_tags.astype(jnp.int32)
    ui = user_interests.astype(jnp.int32)

    run = functools.partial(
        pl.kernel,
        out_type=jax.ShapeDtypeStruct((B,), jnp.float32),
        mesh=plsc.VectorSubcoreMesh(core_axis_name="c", subcore_axis_name="s"),
        compiler_params=pltpu.CompilerParams(
            needs_layout_passes=False, use_tc_tiling_on_sc=False),
        scratch_types=[
            pltpu.VMEM((RPW, H), jnp.int32),
            pltpu.VMEM((RPW, H), jnp.int32),
            pltpu.VMEM((RPW,), jnp.float32),
            [pltpu.VMEM((H, D), jnp.float32) for _ in range(NBUF)],
            [pltpu.VMEM((H, D), jnp.float32) for _ in range(NBUF)],
            [pltpu.SemaphoreType.DMA for _ in range(NBUF)],
        ],
    )(_sc_kernel)

    sim = run(tag_embedding, ct, ui)
    return sim.reshape(B, 1)


# speedup vs baseline: 1.0644x; 1.0644x over previous
import functools

import jax
import jax.numpy as jnp
from jax import lax
from jax.experimental import pallas as pl
from jax.experimental.pallas import tpu as pltpu
from jax.experimental.pallas import tpu_sc as plsc

NUM_TAGS = 100000
D = 32
B = 4096
H = 50
L = 16
NC, NS = 2, 16
NW = NC * NS
RPW = B // NW
NBUF = 8


def _sc_kernel(ct_hbm, ui_hbm, table_hbm, out_hbm,
               cidx, uidx, outv, cbufs, ubufs, semcs, semus):
    wid = lax.axis_index("s") * NC + lax.axis_index("c")
    row0 = wid * RPW

    pltpu.sync_copy(ct_hbm.at[pl.ds(row0, RPW)], cidx)
    pltpu.sync_copy(ui_hbm.at[pl.ds(row0, RPW)], uidx)

    lane = lax.broadcasted_iota(jnp.int32, (L,), 0)
    last_lane = lane == (L - 1)

    def start(j, b):
        pltpu.async_copy(table_hbm.at[cidx.at[j]], cbufs[b], semcs[b])
        pltpu.async_copy(table_hbm.at[uidx.at[j]], ubufs[b], semus[b])

    def wait(b):
        pltpu.make_async_copy(
            table_hbm.at[cidx.at[0]], cbufs[b], semcs[b]).wait()
        pltpu.make_async_copy(
            table_hbm.at[uidx.at[0]], ubufs[b], semus[b]).wait()

    def compute(j, b):
        cb, ub = cbufs[b], ubufs[b]
        c0 = jnp.zeros((L,), jnp.float32)
        c1 = jnp.zeros((L,), jnp.float32)
        u0 = jnp.zeros((L,), jnp.float32)
        u1 = jnp.zeros((L,), jnp.float32)
        for r in range(H):
            c0 = c0 + cb[r, pl.ds(0, L)]
            c1 = c1 + cb[r, pl.ds(L, L)]
            u0 = u0 + ub[r, pl.ds(0, L)]
            u1 = u1 + ub[r, pl.ds(L, L)]
        cs = plsc.cumsum(c0 * u0 + c1 * u1) * (1.0 / (H * H))
        row_idx = jnp.full((L,), j, jnp.int32)
        plsc.store_scatter(outv, [row_idx], cs, mask=last_lane)

    for p in range(NBUF - 1):
        start(p, p)

    @pl.loop(0, RPW, step=NBUF)
    def _chunks(j):
        for b in range(NBUF):
            wait(b)

            @pl.when(j + b + NBUF - 1 < RPW)
            def _():
                start(j + b + NBUF - 1, (b + NBUF - 1) % NBUF)

            compute(j + b, b)

    pltpu.sync_copy(outv, out_hbm.at[pl.ds(row0, RPW)])


@jax.jit
def kernel(course_tags, user_interests, tag_embedding):
    ct = course_tags.astype(jnp.int32)
    ui = user_interests.astype(jnp.int32)

    run = functools.partial(
        pl.kernel,
        out_type=jax.ShapeDtypeStruct((B,), jnp.float32),
        mesh=plsc.VectorSubcoreMesh(core_axis_name="c", subcore_axis_name="s"),
        compiler_params=pltpu.CompilerParams(
            needs_layout_passes=False, use_tc_tiling_on_sc=False),
        scratch_types=[
            pltpu.VMEM((RPW, H), jnp.int32),
            pltpu.VMEM((RPW, H), jnp.int32),
            pltpu.VMEM((RPW,), jnp.float32),
            [pltpu.VMEM((H, D), jnp.float32) for _ in range(NBUF)],
            [pltpu.VMEM((H, D), jnp.float32) for _ in range(NBUF)],
            [pltpu.SemaphoreType.DMA for _ in range(NBUF)],
            [pltpu.SemaphoreType.DMA for _ in range(NBUF)],
        ],
    )(_sc_kernel)

    sim = run(ct, ui, tag_embedding)
    return sim.reshape(B, 1)


# speedup vs baseline: 1.2649x; 1.1884x over previous
import functools

import jax
import jax.numpy as jnp
from jax import lax
from jax.experimental import pallas as pl
from jax.experimental.pallas import tpu as pltpu
from jax.experimental.pallas import tpu_sc as plsc

NUM_TAGS = 100000
D = 32
B = 4096
H = 50
L = 16
NC, NS = 2, 16
NW = NC * NS
RPW = B // NW
NBUF = 4


def _sc_kernel(ct_hbm, ui_hbm, table_hbm, out_hbm,
               cidx, uidx, outv, cbufs, ubufs, semcs, semus):
    wid = lax.axis_index("s") * NC + lax.axis_index("c")
    row0 = wid * RPW

    pltpu.sync_copy(ct_hbm.at[pl.ds(row0, RPW)], cidx)
    pltpu.sync_copy(ui_hbm.at[pl.ds(row0, RPW)], uidx)

    lane = lax.broadcasted_iota(jnp.int32, (L,), 0)
    last_lane = lane == (L - 1)

    def start(j, b):
        pltpu.async_copy(table_hbm.at[cidx.at[j]], cbufs[b], semcs[b])
        pltpu.async_copy(table_hbm.at[uidx.at[j]], ubufs[b], semus[b])

    def wait(b):
        pltpu.make_async_copy(
            table_hbm.at[cidx.at[0]], cbufs[b], semcs[b]).wait()
        pltpu.make_async_copy(
            table_hbm.at[uidx.at[0]], ubufs[b], semus[b]).wait()

    def compute(j, b):
        cb, ub = cbufs[b], ubufs[b]
        c0 = jnp.zeros((L,), jnp.float32)
        c1 = jnp.zeros((L,), jnp.float32)
        u0 = jnp.zeros((L,), jnp.float32)
        u1 = jnp.zeros((L,), jnp.float32)
        for r in range(H):
            c0 = c0 + cb[r, pl.ds(0, L)]
            c1 = c1 + cb[r, pl.ds(L, L)]
            u0 = u0 + ub[r, pl.ds(0, L)]
            u1 = u1 + ub[r, pl.ds(L, L)]
        cs = plsc.cumsum(c0 * u0 + c1 * u1) * (1.0 / (H * H))
        row_idx = jnp.full((L,), j, jnp.int32)
        plsc.store_scatter(outv, [row_idx], cs, mask=last_lane)

    for p in range(NBUF - 1):
        start(p, p)

    @pl.loop(0, RPW, step=NBUF)
    def _chunks(j):
        for b in range(NBUF):
            wait(b)

            @pl.when(j + b + NBUF - 1 < RPW)
            def _():
                start(j + b + NBUF - 1, (b + NBUF - 1) % NBUF)

            compute(j + b, b)

    pltpu.sync_copy(outv, out_hbm.at[pl.ds(row0, RPW)])


@jax.jit
def kernel(course_tags, user_interests, tag_embedding):
    ct = course_tags.astype(jnp.int32)
    ui = user_interests.astype(jnp.int32)

    run = functools.partial(
        pl.kernel,
        out_type=jax.ShapeDtypeStruct((B,), jnp.float32),
        mesh=plsc.VectorSubcoreMesh(core_axis_name="c", subcore_axis_name="s"),
        compiler_params=pltpu.CompilerParams(
            needs_layout_passes=False, use_tc_tiling_on_sc=False),
        scratch_types=[
            pltpu.VMEM((RPW, H), jnp.int32),
            pltpu.VMEM((RPW, H), jnp.int32),
            pltpu.VMEM((RPW,), jnp.float32),
            [pltpu.VMEM((H, D), jnp.float32) for _ in range(NBUF)],
            [pltpu.VMEM((H, D), jnp.float32) for _ in range(NBUF)],
            [pltpu.SemaphoreType.DMA for _ in range(NBUF)],
            [pltpu.SemaphoreType.DMA for _ in range(NBUF)],
        ],
    )(_sc_kernel)

    sim = run(ct, ui, tag_embedding)
    return sim.reshape(B, 1)
